# baseline (speedup 1.0000x reference)
import functools
import jax
import jax.numpy as jnp
from jax import lax
from jax.experimental import pallas as pl
from jax.experimental.pallas import tpu as pltpu
from jax.experimental.pallas import tpu_sc as plsc

N = 50000
E = 800000
B = 128
CW = 128
KB = 80
NTILES = 16
EPT = E // NTILES
NB = EPT // KB
RLEN = 12512
RANGES = [(0, RLEN), (RLEN, RLEN), (2 * RLEN, RLEN), (3 * RLEN, N - 3 * RLEN)]
ZROWS = 784


def _make_segsum(two_chunks):
    nck = 2 if two_chunks else 1
    mesh = plsc.VectorSubcoreMesh(core_axis_name="c", subcore_axis_name="s")

    @functools.partial(
        pl.kernel,
        mesh=mesh,
        out_type=[jax.ShapeDtypeStruct((N, CW), jnp.float32) for _ in range(nck)],
        scratch_types=[
            pltpu.VMEM_SHARED((RLEN + 8, CW), jnp.float32),
            pltpu.VMEM((KB,), jnp.int32),
            pltpu.VMEM((KB,), jnp.int32),
            pltpu.VMEM((KB,), jnp.int32),
            pltpu.VMEM((KB, CW), jnp.float32),
            pltpu.SemaphoreType.DMA,
        ],
    )
    def seg_kernel(*refs):
        h_refs = refs[:nck]
        src_hbm = refs[nck]
        dst_hbm = refs[nck + 1]
        zeros_hbm = refs[nck + 2]
        out_refs = refs[nck + 3:nck + 3 + nck]
        acc, srcv, dstv, idxv, rows, sem = refs[nck + 3 + nck:]

        cid = lax.axis_index("c")
        sid = lax.axis_index("s")
        ebase = sid * EPT

        def range_pass(href, oref, base, vlen):
            trow = 784 * 15
            tlast = vlen - trow

            @pl.when(sid < 15)
            def _():
                pltpu.sync_copy(zeros_hbm.at[pl.ds(0, ZROWS)],
                                acc.at[pl.ds(sid * ZROWS, ZROWS)])

            @pl.when(sid == 15)
            def _():
                pltpu.sync_copy(zeros_hbm.at[pl.ds(0, tlast)],
                                acc.at[pl.ds(trow, tlast)])

            plsc.subcore_barrier()

            def body(j, carry):
                eb = ebase + j * KB
                pltpu.sync_copy(src_hbm.at[pl.ds(eb, KB)], srcv)
                pltpu.sync_copy(dst_hbm.at[pl.ds(eb, KB)], dstv)
                for i in range(KB // 16):
                    d = dstv[pl.ds(i * 16, 16)] - base
                    ok = (d >= 0) & (d < RLEN)
                    idxv[pl.ds(i * 16, 16)] = jnp.where(ok, d, RLEN)
                pltpu.async_copy(href.at[srcv], rows, sem).wait()
                pltpu.sync_copy(rows, acc.at[idxv], add=True)
                return carry

            lax.fori_loop(0, NB, body, 0)
            plsc.subcore_barrier()

            @pl.when(sid < 15)
            def _():
                pltpu.sync_copy(acc.at[pl.ds(sid * ZROWS, ZROWS)],
                                oref.at[pl.ds(base + sid * ZROWS, ZROWS)])

            @pl.when(sid == 15)
            def _():
                pltpu.sync_copy(acc.at[pl.ds(trow, tlast)],
                                oref.at[pl.ds(base + trow, tlast)])

            plsc.subcore_barrier()

        if two_chunks:
            @pl.when(cid == 0)
            def _():
                for base, vlen in RANGES:
                    range_pass(h_refs[0], out_refs[0], base, vlen)

            @pl.when(cid == 1)
            def _():
                for base, vlen in RANGES:
                    range_pass(h_refs[1], out_refs[1], base, vlen)
        else:
            @pl.when(cid == 0)
            def _():
                for base, vlen in RANGES[:2]:
                    range_pass(h_refs[0], out_refs[0], base, vlen)

            @pl.when(cid == 1)
            def _():
                for base, vlen in RANGES[2:]:
                    range_pass(h_refs[0], out_refs[0], base, vlen)

    def run(h, src, dst, zeros):
        if two_chunks:
            o0, o1 = seg_kernel(h[:, :CW], h[:, CW:], src, dst, zeros)
            return jnp.concatenate([o0, o1], axis=1)
        (o,) = seg_kernel(h, src, dst, zeros)
        return o

    return run


_segsum1 = _make_segsum(False)
_segsum2 = _make_segsum(True)

RB = 1000
NRB = N // RB


def _leaky(x):
    return jnp.where(x > 0, x, 0.01 * x)


def _gin_tc_kernel(h_ref, agg_ref, wa_ref, ba_ref, wb_ref, bb_ref,
                   x_ref, s_ref, sq_ref):
    t = h_ref[...] + agg_ref[...]
    u = _leaky(jnp.dot(t.astype(jnp.bfloat16), wa_ref[...].astype(jnp.bfloat16),
                       preferred_element_type=jnp.float32) + ba_ref[...])
    v = jnp.dot(u.astype(jnp.bfloat16), wb_ref[...].astype(jnp.bfloat16),
                preferred_element_type=jnp.float32) + bb_ref[...]
    x = _leaky(v)
    x_ref[...] = x

    @pl.when(pl.program_id(0) == 0)
    def _():
        s_ref[...] = jnp.zeros_like(s_ref)
        sq_ref[...] = jnp.zeros_like(sq_ref)

    s_ref[...] += jnp.sum(x, axis=0, keepdims=True)
    sq_ref[...] += jnp.sum(x * x, axis=0, keepdims=True)


def _gin_tc(h, agg, wa, ba, wb, bb):
    din = h.shape[1]
    hdim = wa.shape[1]
    return pl.pallas_call(
        _gin_tc_kernel,
        grid=(NRB,),
        in_specs=[
            pl.BlockSpec((RB, din), lambda i: (i, 0)),
            pl.BlockSpec((RB, din), lambda i: (i, 0)),
            pl.BlockSpec((din, hdim), lambda i: (0, 0)),
            pl.BlockSpec((1, hdim), lambda i: (0, 0)),
            pl.BlockSpec((hdim, hdim), lambda i: (0, 0)),
            pl.BlockSpec((1, hdim), lambda i: (0, 0)),
        ],
        out_specs=[
            pl.BlockSpec((RB, hdim), lambda i: (i, 0)),
            pl.BlockSpec((1, hdim), lambda i: (0, 0)),
            pl.BlockSpec((1, hdim), lambda i: (0, 0)),
        ],
        out_shape=[
            jax.ShapeDtypeStruct((N, hdim), jnp.float32),
            jax.ShapeDtypeStruct((1, hdim), jnp.float32),
            jax.ShapeDtypeStruct((1, hdim), jnp.float32),
        ],
    )(h, agg, wa, ba.reshape(1, -1), wb, bb.reshape(1, -1))


def _bn_apply_kernel(x_ref, mu_ref, var_ref, g_ref, be_ref, o_ref):
    o_ref[...] = ((x_ref[...] - mu_ref[...]) / jnp.sqrt(var_ref[...] + 1e-5)
                  * g_ref[...] + be_ref[...])


def _bn_apply(x, mu, var, g, be):
    hdim = x.shape[1]
    return pl.pallas_call(
        _bn_apply_kernel,
        grid=(NRB,),
        in_specs=[
            pl.BlockSpec((RB, hdim), lambda i: (i, 0)),
            pl.BlockSpec((1, hdim), lambda i: (0, 0)),
            pl.BlockSpec((1, hdim), lambda i: (0, 0)),
            pl.BlockSpec((1, hdim), lambda i: (0, 0)),
            pl.BlockSpec((1, hdim), lambda i: (0, 0)),
        ],
        out_specs=pl.BlockSpec((RB, hdim), lambda i: (i, 0)),
        out_shape=jax.ShapeDtypeStruct(x.shape, jnp.float32),
    )(x, mu.reshape(1, -1), var.reshape(1, -1), g.reshape(1, -1),
      be.reshape(1, -1))


def _jk_pool_kernel(x1_ref, x2_ref, x3_ref, b_ref, xj_ref, pooled_ref):
    xj = jnp.maximum(jnp.maximum(x1_ref[...], x2_ref[...]), x3_ref[...])
    xj_ref[...] = xj

    bvec = b_ref[0, 0, :].reshape(RB, 1)
    ids = lax.broadcasted_iota(jnp.int32, (RB, B), 1)
    oh = (bvec == ids).astype(jnp.float32)
    contrib = lax.dot_general(oh, xj, (((0,), (0,)), ((), ())),
                              preferred_element_type=jnp.float32,
                              precision=lax.Precision.HIGHEST)

    @pl.when(pl.program_id(0) == 0)
    def _():
        pooled_ref[...] = jnp.zeros_like(pooled_ref)

    pooled_ref[...] += contrib


def _jk_pool(x1b, x2b, x3, batch):
    hdim = x1b.shape[1]
    b3 = batch.reshape(NRB, 1, RB)
    return pl.pallas_call(
        _jk_pool_kernel,
        grid=(NRB,),
        in_specs=[
            pl.BlockSpec((RB, hdim), lambda i: (i, 0)),
            pl.BlockSpec((RB, hdim), lambda i: (i, 0)),
            pl.BlockSpec((RB, hdim), lambda i: (i, 0)),
            pl.BlockSpec((1, 1, RB), lambda i: (i, 0, 0)),
        ],
        out_specs=[
            pl.BlockSpec((RB, hdim), lambda i: (i, 0)),
            pl.BlockSpec((B, hdim), lambda i: (0, 0)),
        ],
        out_shape=[
            jax.ShapeDtypeStruct((N, hdim), jnp.float32),
            jax.ShapeDtypeStruct((B, hdim), jnp.float32),
        ],
    )(x1b, x2b, x3, b3)


def _head_kernel(p_ref, w1_ref, b1_ref, w2_ref, b2_ref, w3_ref, b3_ref, o_ref):
    o1 = jnp.dot(p_ref[...], w1_ref[...], preferred_element_type=jnp.float32, precision=lax.Precision.HIGHEST) + b1_ref[...]
    o2 = _leaky(jnp.dot(o1, w2_ref[...], preferred_element_type=jnp.float32, precision=lax.Precision.HIGHEST) + b2_ref[...])
    o_ref[...] = jnp.dot(o2, w3_ref[...], preferred_element_type=jnp.float32, precision=lax.Precision.HIGHEST) + b3_ref[...]


def _head(pooled, w1, b1, w2, b2, w3p, b3p):
    return pl.pallas_call(
        _head_kernel,
        out_shape=jax.ShapeDtypeStruct((B, 128), jnp.float32),
    )(pooled, w1, b1.reshape(1, -1), w2, b2.reshape(1, -1), w3p, b3p.reshape(1, -1))


def _bn_params(s, sq, g, be):
    mu = s[0] / N
    var = jnp.maximum(sq[0] / N - mu * mu, 0.0)
    scale = g * lax.rsqrt(var + 1e-5)
    shift = be - mu * scale
    return scale, shift


@jax.jit
def _impl(x, edge_index, batch, W1a, b1a, W1b, b1b, g1, be1,
          W2a, b2a, W2b, b2b, g2, be2, W3a, b3a, W3b, b3b,
          Wf1, bf1, Wf2, bf2, Wf3, bf3):
    src = edge_index[0]
    dst = edge_index[1]
    zeros = jnp.zeros((ZROWS, CW), jnp.float32)

    xpad = jnp.pad(x, ((0, 0), (0, CW - x.shape[1])))
    agg1 = _segsum1(xpad, src, dst, zeros)[:, :x.shape[1]]
    x1, _, _ = _gin_tc(x, agg1, W1a, b1a, W1b, b1b)
    mu1 = x1.mean(axis=0)
    var1 = jnp.mean((x1 - mu1) ** 2, axis=0)
    x1b = _bn_apply(x1, mu1, var1, g1, be1)

    agg2 = _segsum2(x1b, src, dst, zeros)
    x2, _, _ = _gin_tc(x1b, agg2, W2a, b2a, W2b, b2b)
    mu2 = x2.mean(axis=0)
    var2 = jnp.mean((x2 - mu2) ** 2, axis=0)
    x2b = _bn_apply(x2, mu2, var2, g2, be2)

    agg3 = _segsum2(x2b, src, dst, zeros)
    x3, _, _ = _gin_tc(x2b, agg3, W3a, b3a, W3b, b3b)

    xj, pooled = _jk_pool(x1b, x2b, x3, batch)

    counts = jnp.bincount(batch, length=B)
    starts = jnp.cumsum(counts) - counts
    order = jnp.lexsort((-xj[:, -1], batch))
    sorted_batch = batch[order]
    pos_sorted = (order - starts[sorted_batch]).astype(jnp.int32)
    rank = jnp.arange(N, dtype=jnp.int32) - starts[sorted_batch].astype(jnp.int32)
    sel = jnp.broadcast_to(jnp.arange(20, dtype=jnp.int32), (B, 20))
    sel = sel.at[sorted_batch, rank].set(pos_sorted, mode='drop')
    select_index = sel.astype(jnp.float32)

    w3p = jnp.zeros((64, 128), jnp.float32).at[:, :10].set(Wf3)
    b3p = jnp.zeros((128,), jnp.float32).at[:10].set(bf3)
    o = _head(pooled, Wf1, bf1, Wf2, bf2, w3p, b3p)[:, :10]
    return (o, select_index)


def kernel(x, edge_index, batch, W1a, b1a, W1b, b1b, g1, be1,
           W2a, b2a, W2b, b2b, g2, be2, W3a, b3a, W3b, b3b,
           Wf1, bf1, Wf2, bf2, Wf3, bf3):
    return _impl(x, edge_index, batch, W1a, b1a, W1b, b1b, g1, be1,
                 W2a, b2a, W2b, b2b, g2, be2, W3a, b3a, W3b, b3b,
                 Wf1, bf1, Wf2, bf2, Wf3, bf3)
